# baseline (speedup 1.0000x reference)
import jax
import jax.numpy as jnp
from jax.experimental import pallas as pl
from jax.experimental.pallas import tpu as pltpu

_B = 512
_K = 1024
_D = 256
_GRID = 32
_KB = 256
_NB = _K // _KB

_MAX_EPOCHS = 100
_INITIAL_RADIUS = 16.0
_INITIAL_LR = 0.1
_STD_COEFF = 0.5


def _som_kernel(x_ref, w_ref, epoch_ref, locs_ref, nw_ref,
                xt_ref, m_ref, idx_ref):
    pid = pl.program_id(0)

    @pl.when(pid == 0)
    def _prologue():
        xt_ref[...] = x_ref[...].T
        m_ref[...] = jnp.full((1, _B), jnp.inf, dtype=jnp.float32)
        idx_ref[...] = jnp.full((1, _B), _K, dtype=jnp.int32)

    @pl.when(pid < _NB)
    def _distance_block():
        w_b = w_ref[...]
        xt = xt_ref[...]
        wsq = jnp.sum(w_b * w_b, axis=1, keepdims=True)
        wx = jax.lax.dot_general(
            w_b, xt, (((1,), (0,)), ((), ())),
            preferred_element_type=jnp.float32,
            precision=jax.lax.Precision.HIGHEST,
        )
        d = wsq - 2.0 * wx
        bm = jnp.min(d, axis=0, keepdims=True)
        kio_b = (jax.lax.broadcasted_iota(jnp.int32, (_KB, _B), 0)
                 + pid * _KB)
        bidx = jnp.min(jnp.where(d == bm, kio_b, jnp.int32(_K)), axis=0,
                       keepdims=True)
        upd = bm < m_ref[...]
        idx_ref[...] = jnp.where(upd, bidx, idx_ref[...])
        m_ref[...] = jnp.where(upd, bm, m_ref[...])

    @pl.when(pid == _NB)
    def _update():
        epoch_f = epoch_ref[0]
        radius = _INITIAL_RADIUS - epoch_f * (
            (_INITIAL_RADIUS - 1.0) / float(_MAX_EPOCHS - 1))
        alpha = _INITIAL_LR * (1.0 - epoch_f / float(_MAX_EPOCHS))
        sigma = radius * _STD_COEFF
        neg_inv_two_sigma_sq = -1.0 / (2.0 * sigma * sigma)

        idx_t = idx_ref[...]
        idx = idx_t.reshape(_B, 1)
        locs_ref[...] = jnp.concatenate(
            [idx >> 5, idx & (_GRID - 1)], axis=1)

        bi = idx_t >> 5
        bj = idx_t & (_GRID - 1)
        kio_t = jax.lax.broadcasted_iota(jnp.int32, (_K, _B), 0)
        li = kio_t >> 5
        lj = kio_t & (_GRID - 1)
        d2 = (li - bi) * (li - bi) + (lj - bj) * (lj - bj)
        lr_t = alpha * jnp.exp(d2.astype(jnp.float32)
                               * neg_inv_two_sigma_sq)

        den = jnp.sum(lr_t, axis=1, keepdims=True) + 1e-12
        x = x_ref[...]
        num = jax.lax.dot_general(
            lr_t.astype(jnp.bfloat16), x.astype(jnp.bfloat16),
            (((1,), (0,)), ((), ())),
            preferred_element_type=jnp.float32,
        )
        nw_ref[...] = num / den


def kernel(input_vect, weights, epoch):
    epoch_f = jnp.asarray(epoch, dtype=jnp.float32).reshape(1)
    return pl.pallas_call(
        _som_kernel,
        grid=(_NB + 1,),
        out_shape=[
            jax.ShapeDtypeStruct((_B, 2), jnp.int32),
            jax.ShapeDtypeStruct((_K, _D), jnp.float32),
        ],
        in_specs=[
            pl.BlockSpec((_B, _D), lambda i: (0, 0),
                         memory_space=pltpu.VMEM),
            pl.BlockSpec((_KB, _D),
                         lambda i: (jnp.minimum(i, _NB - 1), 0),
                         memory_space=pltpu.VMEM),
            pl.BlockSpec(memory_space=pltpu.SMEM),
        ],
        out_specs=[
            pl.BlockSpec((_B, 2), lambda i: (0, 0),
                         memory_space=pltpu.VMEM),
            pl.BlockSpec((_K, _D), lambda i: (0, 0),
                         memory_space=pltpu.VMEM),
        ],
        scratch_shapes=[
            pltpu.VMEM((_D, _B), jnp.float32),
            pltpu.VMEM((1, _B), jnp.float32),
            pltpu.VMEM((1, _B), jnp.int32),
        ],
    )(input_vect, weights, epoch_f)
